# Initial kernel scaffold; baseline (speedup 1.0000x reference)
import jax
import jax.numpy as jnp
from jax.experimental import pallas as pl


def kernel(feats, preds, W_proj, b_proj):
    raise NotImplementedError("write your pallas kernel here")


# speedup vs baseline: 1.4931x; 1.4931x over previous
import functools

import jax
import jax.numpy as jnp
from jax.experimental import pallas as pl

DECAY = 0.9
LANE_N = 6272


def _proj_pool_body(feats_ref, preds_ref, w_ref, bias_ref, out_ref, acc_ref):
    b = pl.program_id(0)
    n = pl.program_id(1)
    f = feats_ref[0]
    w = w_ref[...]
    p = jnp.dot(w, f, preferred_element_type=jnp.float32)
    p = p + bias_ref[0][:, None]
    out_ref[0] = p
    m = preds_ref[0]
    contrib = jnp.sum(p * m, axis=1)

    @pl.when(n == 0)
    def _init():
        acc_ref[0, 0] = jnp.zeros_like(acc_ref[0, 0])

    acc_ref[0, 0] = acc_ref[0, 0] + contrib


def _memory_update_body(acc_ref, mem_ref, mask_ref, *, hw, nslots):
    pooled = acc_ref[:, 0, :] / float(hw)
    B, C = pooled.shape
    mem = jnp.zeros((B, C), dtype=jnp.float32)
    slotv = jax.lax.broadcasted_iota(jnp.int32, (B, 1), 0)
    ptr = jnp.int32(0)
    for i in range(nslots):
        v = pooled[i:i + 1, :]
        vn = v / jnp.sqrt(jnp.sum(v * v))
        mn = mem / jnp.sqrt(jnp.sum(mem * mem, axis=1, keepdims=True))
        cos = jnp.sum(mn * vn, axis=1, keepdims=True)
        cosm = jnp.where(slotv < ptr, cos, -1e30)
        val = jnp.max(cosm)
        idx = jnp.min(jnp.where(cosm == val, slotv, B))
        do_merge = val >= 0.5
        target = jnp.where(do_merge, idx, ptr)
        mrow = jnp.sum(jnp.where(slotv == idx, mem, 0.0), axis=0, keepdims=True)
        newrow = jnp.where(do_merge, mrow * DECAY + (1.0 - DECAY) * v, v)
        mem = jnp.where(slotv == target, newrow, mem)
        ptr = ptr + jnp.where(do_merge, jnp.int32(0), jnp.int32(1))
    mem_ref[...] = mem
    mask_ref[...] = jnp.where(slotv < ptr, 0.0, -1e30) + jnp.zeros((1, C))


def _attend_body(big_ref, mem_ref, mask_ref, out_ref):
    proj = big_ref[0]
    mem = mem_ref[...]
    logit = jnp.dot(mem, proj, preferred_element_type=jnp.float32)
    logit = logit + mask_ref[:, :1]
    mx = jnp.max(logit, axis=0, keepdims=True)
    e = jnp.exp(logit - mx)
    attn = e / jnp.sum(e, axis=0, keepdims=True)
    aug = jnp.dot(mem.T, attn, preferred_element_type=jnp.float32)
    out_ref[0] = aug


def kernel(feats, preds, W_proj, b_proj):
    B, Cin, H, W = feats.shape
    Cout = W_proj.shape[0]
    HW = H * W
    N = LANE_N
    T = HW // N

    feats_r = feats.reshape(B, Cin, HW)
    preds_r = preds.reshape(B, 1, HW)
    bias_r = b_proj.reshape(1, Cout)

    big, acc = pl.pallas_call(
        _proj_pool_body,
        grid=(B, T),
        in_specs=[
            pl.BlockSpec((1, Cin, N), lambda b, n: (b, 0, n)),
            pl.BlockSpec((1, 1, N), lambda b, n: (b, 0, n)),
            pl.BlockSpec((Cout, Cin), lambda b, n: (0, 0)),
            pl.BlockSpec((1, Cout), lambda b, n: (0, 0)),
        ],
        out_specs=[
            pl.BlockSpec((1, Cout, N), lambda b, n: (b, 0, n)),
            pl.BlockSpec((1, 1, Cout), lambda b, n: (b, 0, 0)),
        ],
        out_shape=[
            jax.ShapeDtypeStruct((B, 2 * Cout, HW), jnp.float32),
            jax.ShapeDtypeStruct((B, 1, Cout), jnp.float32),
        ],
    )(feats_r, preds_r, W_proj, bias_r)

    mem, mask = pl.pallas_call(
        functools.partial(_memory_update_body, hw=HW, nslots=B),
        out_shape=[
            jax.ShapeDtypeStruct((B, Cout), jnp.float32),
            jax.ShapeDtypeStruct((B, Cout), jnp.float32),
        ],
    )(acc)

    out = pl.pallas_call(
        _attend_body,
        grid=(B, T),
        in_specs=[
            pl.BlockSpec((1, Cout, N), lambda b, n: (b, 0, n)),
            pl.BlockSpec((B, Cout), lambda b, n: (0, 0)),
            pl.BlockSpec((B, Cout), lambda b, n: (0, 0)),
        ],
        out_specs=pl.BlockSpec((1, Cout, N), lambda b, n: (b, 1, n)),
        out_shape=jax.ShapeDtypeStruct((B, 2 * Cout, HW), jnp.float32),
        input_output_aliases={0: 0},
    )(big, mem, mask)

    return out.reshape(B, 2 * Cout, H, W)
